# baseline (speedup 1.0000x reference)
import functools

import jax
import jax.numpy as jnp
from jax import lax
from jax.experimental import pallas as pl
from jax.experimental.pallas import tpu as pltpu
from jax.experimental.pallas import tpu_sc as plsc

N_NODES = 10000
N_EDGES = 160000
NC, NS = 2, 16
NW = NC * NS
CHUNK_BF, NCH_BF = 128, 80
CHUNK_F32, NCH_F32 = 112, 90
ACC_ROWS = 10016
MBLK = 1000


def _spmm(table, gidx, didx, init, feat, dtype, chunk, nch):
    mesh = plsc.VectorSubcoreMesh(core_axis_name="c", subcore_axis_name="s")
    ri = ACC_ROWS // NS
    ro = N_NODES // NS
    NCH, CHUNK = nch, chunk

    @functools.partial(
        pl.kernel,
        out_type=jax.ShapeDtypeStruct((2 * N_NODES, feat), dtype),
        mesh=mesh,
        scratch_types=[
            pltpu.VMEM((NCH, CHUNK), jnp.int32),
            pltpu.VMEM((NCH, CHUNK), jnp.int32),
            pltpu.VMEM((CHUNK, feat), dtype),
            pltpu.VMEM((CHUNK, feat), dtype),
            pltpu.VMEM_SHARED((ACC_ROWS, feat), dtype),
            pltpu.SemaphoreType.DMA,
            pltpu.SemaphoreType.DMA,
            pltpu.SemaphoreType.DMA,
            pltpu.SemaphoreType.DMA,
        ],
        compiler_params=pltpu.CompilerParams(use_tc_tiling_on_sc=False),
    )
    def spmm(table_hbm, gidx_hbm, didx_hbm, init_hbm, out_hbm,
             gi_v, di_v, buf0, buf1, acc, semg0, semg1, sems0, sems1):
        c = lax.axis_index("c")
        s = lax.axis_index("s")
        pltpu.sync_copy(gidx_hbm.at[s], gi_v)
        pltpu.sync_copy(didx_hbm.at[s], di_v)
        pltpu.sync_copy(init_hbm.at[c, pl.ds(s * ri, ri)],
                        acc.at[pl.ds(s * ri, ri)])

        @pl.when(c == 1)
        def _():
            def bias_row(j, carry):
                for k in range(CHUNK // 16):
                    gi_v[j, pl.ds(k * 16, 16)] = (
                        gi_v[j, pl.ds(k * 16, 16)] + N_NODES)
                return carry
            lax.fori_loop(0, NCH, bias_row, 0)
        plsc.subcore_barrier()

        pltpu.async_copy(table_hbm.at[gi_v.at[0]], buf0, semg0)
        pltpu.async_copy(table_hbm.at[gi_v.at[1]], buf1, semg1)

        def body(i, carry):
            a = 2 * i
            pltpu.make_async_copy(table_hbm.at[gi_v.at[a]], buf0, semg0).wait()
            pltpu.sync_copy(buf0, acc.at[di_v.at[a]], add=True)

            @pl.when(i < NCH // 2 - 1)
            def _():
                pltpu.async_copy(table_hbm.at[gi_v.at[a + 2]], buf0, semg0)
            pltpu.make_async_copy(table_hbm.at[gi_v.at[a + 1]], buf1, semg1).wait()
            pltpu.sync_copy(buf1, acc.at[di_v.at[a + 1]], add=True)

            @pl.when(i < NCH // 2 - 1)
            def _():
                pltpu.async_copy(table_hbm.at[gi_v.at[a + 3]], buf1, semg1)
            return carry
        lax.fori_loop(0, NCH // 2, body, 0)
        plsc.subcore_barrier()

        pltpu.sync_copy(acc.at[pl.ds(s * ro, ro)],
                        out_hbm.at[pl.ds(c * N_NODES + s * ro, ro)])

    return spmm(table, gidx, didx, init)


def _mm0(x, w):
    def body(x_ref, w_ref, o_ref):
        o_ref[...] = jnp.dot(x_ref[...], w_ref[...],
                             preferred_element_type=jnp.float32
                             ).astype(jnp.bfloat16)
    return pl.pallas_call(
        body,
        grid=(N_NODES // MBLK, 2),
        in_specs=[pl.BlockSpec((MBLK, 256), lambda i, j: (i, 0)),
                  pl.BlockSpec((256, 128), lambda i, j: (0, j))],
        out_specs=pl.BlockSpec((MBLK, 128), lambda i, j: (j * (N_NODES // MBLK) + i, 0)),
        out_shape=jax.ShapeDtypeStruct((2 * N_NODES, 128), jnp.bfloat16),
    )(x, w)


def _mm1(hst, w):
    nb = N_NODES // MBLK

    def body(hlo_ref, hhi_ref, w_ref, o_ref):
        hlo = jnp.maximum(hlo_ref[...].astype(jnp.float32), 0.0)
        hhi = jnp.maximum(hhi_ref[...].astype(jnp.float32), 0.0)
        part = (jnp.dot(hlo, w_ref[:128], preferred_element_type=jnp.float32)
                + jnp.dot(hhi, w_ref[128:], preferred_element_type=jnp.float32))
        o_ref[...] = part.astype(jnp.bfloat16)

    return pl.pallas_call(
        body,
        grid=(nb, 2),
        in_specs=[pl.BlockSpec((MBLK, 128), lambda i, j: (i, 0)),
                  pl.BlockSpec((MBLK, 128), lambda i, j: (nb + i, 0)),
                  pl.BlockSpec((256, 128), lambda i, j: (0, j))],
        out_specs=pl.BlockSpec((MBLK, 128), lambda i, j: (j * nb + i, 0)),
        out_shape=jax.ShapeDtypeStruct((2 * N_NODES, 128), jnp.bfloat16),
    )(hst, hst, w)


def _colsum_relu(hst):
    nb = N_NODES // MBLK

    def body(a_ref, b_ref, o_ref):
        i = pl.program_id(0)
        a = jnp.maximum(a_ref[...].astype(jnp.float32), 0.0)
        b = jnp.maximum(b_ref[...].astype(jnp.float32), 0.0)
        sa = jnp.sum(a, axis=0, keepdims=True)
        sb = jnp.sum(b, axis=0, keepdims=True)
        part = jnp.concatenate([sa, sb], axis=0)

        @pl.when(i == 0)
        def _():
            o_ref[...] = part

        @pl.when(i != 0)
        def _():
            o_ref[...] += part

    return pl.pallas_call(
        body,
        grid=(nb,),
        in_specs=[pl.BlockSpec((MBLK, 128), lambda i: (i, 0)),
                  pl.BlockSpec((MBLK, 128), lambda i: (nb + i, 0))],
        out_specs=pl.BlockSpec((2, 128), lambda i: (0, 0)),
        out_shape=jax.ShapeDtypeStruct((2, 128), jnp.float32),
    )(hst, hst)


def _mm2(hst, csum, w2b):
    nb = N_NODES // MBLK

    def body(hlo_ref, hhi_ref, c_ref, wlo_ref, whi_ref, o_ref):
        m = c_ref[...] * (1.0 / N_NODES)
        hlo = jnp.maximum(hlo_ref[...].astype(jnp.float32), 0.0) + m[0:1]
        hhi = jnp.maximum(hhi_ref[...].astype(jnp.float32), 0.0) + m[1:2]
        o_ref[...] = (
            jnp.dot(hlo, wlo_ref[0, 0], preferred_element_type=jnp.float32)
            + jnp.dot(hhi, whi_ref[0, 0], preferred_element_type=jnp.float32))

    return pl.pallas_call(
        body,
        grid=(nb, 2),
        in_specs=[pl.BlockSpec((MBLK, 128), lambda i, j: (i, 0)),
                  pl.BlockSpec((MBLK, 128), lambda i, j: (nb + i, 0)),
                  pl.BlockSpec((2, 128), lambda i, j: (0, 0)),
                  pl.BlockSpec((1, 1, 128, 32), lambda i, j: (0, j, 0, 0)),
                  pl.BlockSpec((1, 1, 128, 32), lambda i, j: (1, j, 0, 0))],
        out_specs=pl.BlockSpec((MBLK, 32), lambda i, j: (j * nb + i, 0)),
        out_shape=jax.ShapeDtypeStruct((2 * N_NODES, 32), jnp.float32),
    )(hst, hst, csum, w2b, w2b)


def _edge_layout(src, dst, chunk, nch):
    e_pad = NS * nch * chunk
    pad = e_pad - N_EDGES
    src_p = jnp.concatenate([src, jnp.zeros((pad,), jnp.int32)])
    dst_p = jnp.concatenate([dst, jnp.full((pad,), N_NODES, jnp.int32)])
    return src_p.reshape(NS, nch, chunk), dst_p.reshape(NS, nch, chunk)


def kernel(x, edge_index, W0, W1, W2, b2):
    src = edge_index[0].astype(jnp.int32)
    dst = edge_index[1].astype(jnp.int32)
    gidx_bf, didx_bf = _edge_layout(src, dst, CHUNK_BF, NCH_BF)
    gidx_f, didx_f = _edge_layout(src, dst, CHUNK_F32, NCH_F32)
    zinit = jnp.zeros((2, ACC_ROWS, 128), jnp.bfloat16)
    binit = jnp.broadcast_to(b2.reshape(2, 1, 32), (2, ACC_ROWS, 32))

    s0 = _mm0(x, W0)
    h1 = _spmm(s0, gidx_bf, didx_bf, zinit, 128, jnp.bfloat16, CHUNK_BF, NCH_BF)
    s1 = _mm1(h1, W1)
    h2 = _spmm(s1, gidx_bf, didx_bf, zinit, 128, jnp.bfloat16, CHUNK_BF, NCH_BF)
    cs = _colsum_relu(h2)
    w2b = W2.reshape(2, 128, 2, 32).transpose(0, 2, 1, 3)
    s2 = _mm2(h2, cs, w2b)
    o = _spmm(s2, gidx_f, didx_f, binit, 32, jnp.float32, CHUNK_F32, NCH_F32)
    return jnp.concatenate([o[:N_NODES], o[N_NODES:]], axis=1)


# speedup vs baseline: 1.2503x; 1.2503x over previous
import functools

import jax
import jax.numpy as jnp
from jax import lax
from jax.experimental import pallas as pl
from jax.experimental.pallas import tpu as pltpu
from jax.experimental.pallas import tpu_sc as plsc

N_NODES = 10000
N_EDGES = 160000
NC, NS = 2, 16
NW = NC * NS
CHUNK_BF, NCH_BF = 128, 80
CHUNK_F32, NCH_F32 = 112, 90
ACC_ROWS = 10016
MBLK = 1000


def _spmm(table, gidx, didx, init, feat, dtype, chunk, nch):
    mesh = plsc.VectorSubcoreMesh(core_axis_name="c", subcore_axis_name="s")
    ri = ACC_ROWS // NS
    ro = N_NODES // NS
    NCH, CHUNK = nch, chunk

    @functools.partial(
        pl.kernel,
        out_type=jax.ShapeDtypeStruct((2 * N_NODES, feat), dtype),
        mesh=mesh,
        scratch_types=[
            pltpu.VMEM((NCH, CHUNK), jnp.int32),
            pltpu.VMEM((NCH, CHUNK), jnp.int32),
            pltpu.VMEM((CHUNK, feat), dtype),
            pltpu.VMEM((CHUNK, feat), dtype),
            pltpu.VMEM_SHARED((ACC_ROWS, feat), dtype),
            pltpu.SemaphoreType.DMA,
            pltpu.SemaphoreType.DMA,
            pltpu.SemaphoreType.DMA,
            pltpu.SemaphoreType.DMA,
        ],
        compiler_params=pltpu.CompilerParams(use_tc_tiling_on_sc=False),
    )
    def spmm(table_hbm, gidx_hbm, didx_hbm, init_hbm, out_hbm,
             gi_v, di_v, buf0, buf1, acc, semg0, semg1, sems0, sems1):
        c = lax.axis_index("c")
        s = lax.axis_index("s")
        pltpu.sync_copy(gidx_hbm.at[s], gi_v)
        pltpu.sync_copy(didx_hbm.at[s], di_v)
        pltpu.sync_copy(init_hbm.at[c, pl.ds(s * ri, ri)],
                        acc.at[pl.ds(s * ri, ri)])

        @pl.when(c == 1)
        def _():
            def bias_row(j, carry):
                for k in range(CHUNK // 16):
                    gi_v[j, pl.ds(k * 16, 16)] = (
                        gi_v[j, pl.ds(k * 16, 16)] + N_NODES)
                return carry
            lax.fori_loop(0, NCH, bias_row, 0)
        plsc.subcore_barrier()

        pltpu.async_copy(table_hbm.at[gi_v.at[0]], buf0, semg0)
        pltpu.async_copy(table_hbm.at[gi_v.at[1]], buf1, semg1)

        def body(i, carry):
            a = 2 * i
            pltpu.make_async_copy(table_hbm.at[gi_v.at[a]], buf0, semg0).wait()
            pltpu.sync_copy(buf0, acc.at[di_v.at[a]], add=True)

            @pl.when(i < NCH // 2 - 1)
            def _():
                pltpu.async_copy(table_hbm.at[gi_v.at[a + 2]], buf0, semg0)
            pltpu.make_async_copy(table_hbm.at[gi_v.at[a + 1]], buf1, semg1).wait()
            pltpu.sync_copy(buf1, acc.at[di_v.at[a + 1]], add=True)

            @pl.when(i < NCH // 2 - 1)
            def _():
                pltpu.async_copy(table_hbm.at[gi_v.at[a + 3]], buf1, semg1)
            return carry
        lax.fori_loop(0, NCH // 2, body, 0)
        plsc.subcore_barrier()

        pltpu.sync_copy(acc.at[pl.ds(s * ro, ro)],
                        out_hbm.at[pl.ds(c * N_NODES + s * ro, ro)])

    return spmm(table, gidx, didx, init)


def _mm0(x, w):
    def body(x_ref, w_ref, o_ref):
        o_ref[...] = jnp.dot(x_ref[...], w_ref[...],
                             preferred_element_type=jnp.float32)
    return pl.pallas_call(
        body,
        grid=(N_NODES // MBLK, 2),
        in_specs=[pl.BlockSpec((MBLK, 256), lambda i, j: (i, 0)),
                  pl.BlockSpec((256, 128), lambda i, j: (0, j))],
        out_specs=pl.BlockSpec((MBLK, 128), lambda i, j: (j * (N_NODES // MBLK) + i, 0)),
        out_shape=jax.ShapeDtypeStruct((2 * N_NODES, 128), jnp.float32),
    )(x, w)


def _mm1(hst, w):
    nb = N_NODES // MBLK

    def body(hlo_ref, hhi_ref, w_ref, o_ref):
        hlo = jnp.maximum(hlo_ref[...].astype(jnp.float32), 0.0)
        hhi = jnp.maximum(hhi_ref[...].astype(jnp.float32), 0.0)
        part = (jnp.dot(hlo, w_ref[:128], preferred_element_type=jnp.float32)
                + jnp.dot(hhi, w_ref[128:], preferred_element_type=jnp.float32))
        o_ref[...] = part

    return pl.pallas_call(
        body,
        grid=(nb, 2),
        in_specs=[pl.BlockSpec((MBLK, 128), lambda i, j: (i, 0)),
                  pl.BlockSpec((MBLK, 128), lambda i, j: (nb + i, 0)),
                  pl.BlockSpec((256, 128), lambda i, j: (0, j))],
        out_specs=pl.BlockSpec((MBLK, 128), lambda i, j: (j * nb + i, 0)),
        out_shape=jax.ShapeDtypeStruct((2 * N_NODES, 128), jnp.float32),
    )(hst, hst, w)


def _colsum_relu(hst):
    nb = N_NODES // MBLK

    def body(a_ref, b_ref, o_ref):
        i = pl.program_id(0)
        a = jnp.maximum(a_ref[...].astype(jnp.float32), 0.0)
        b = jnp.maximum(b_ref[...].astype(jnp.float32), 0.0)
        sa = jnp.sum(a, axis=0, keepdims=True)
        sb = jnp.sum(b, axis=0, keepdims=True)
        part = jnp.concatenate([sa, sb], axis=0)

        @pl.when(i == 0)
        def _():
            o_ref[...] = part

        @pl.when(i != 0)
        def _():
            o_ref[...] += part

    return pl.pallas_call(
        body,
        grid=(nb,),
        in_specs=[pl.BlockSpec((MBLK, 128), lambda i: (i, 0)),
                  pl.BlockSpec((MBLK, 128), lambda i: (nb + i, 0))],
        out_specs=pl.BlockSpec((2, 128), lambda i: (0, 0)),
        out_shape=jax.ShapeDtypeStruct((2, 128), jnp.float32),
    )(hst, hst)


def _mm2(hst, csum, w2b):
    nb = N_NODES // MBLK

    def body(hlo_ref, hhi_ref, c_ref, wlo_ref, whi_ref, o_ref):
        m = c_ref[...] * (1.0 / N_NODES)
        hlo = jnp.maximum(hlo_ref[...].astype(jnp.float32), 0.0) + m[0:1]
        hhi = jnp.maximum(hhi_ref[...].astype(jnp.float32), 0.0) + m[1:2]
        o_ref[...] = (
            jnp.dot(hlo, wlo_ref[0, 0], preferred_element_type=jnp.float32)
            + jnp.dot(hhi, whi_ref[0, 0], preferred_element_type=jnp.float32))

    return pl.pallas_call(
        body,
        grid=(nb, 2),
        in_specs=[pl.BlockSpec((MBLK, 128), lambda i, j: (i, 0)),
                  pl.BlockSpec((MBLK, 128), lambda i, j: (nb + i, 0)),
                  pl.BlockSpec((2, 128), lambda i, j: (0, 0)),
                  pl.BlockSpec((1, 1, 128, 32), lambda i, j: (0, j, 0, 0)),
                  pl.BlockSpec((1, 1, 128, 32), lambda i, j: (1, j, 0, 0))],
        out_specs=pl.BlockSpec((MBLK, 32), lambda i, j: (j * nb + i, 0)),
        out_shape=jax.ShapeDtypeStruct((2 * N_NODES, 32), jnp.float32),
    )(hst, hst, csum, w2b, w2b)


def _edge_layout(src, dst, chunk, nch):
    e_pad = NS * nch * chunk
    pad = e_pad - N_EDGES
    src_p = jnp.concatenate([src, jnp.zeros((pad,), jnp.int32)])
    dst_p = jnp.concatenate([dst, jnp.full((pad,), N_NODES, jnp.int32)])
    return src_p.reshape(NS, nch, chunk), dst_p.reshape(NS, nch, chunk)


def kernel(x, edge_index, W0, W1, W2, b2):
    src = edge_index[0].astype(jnp.int32)
    dst = edge_index[1].astype(jnp.int32)
    gidx_f, didx_f = _edge_layout(src, dst, CHUNK_F32, NCH_F32)
    zinit = jnp.zeros((2, ACC_ROWS, 128), jnp.float32)
    binit = jnp.broadcast_to(b2.reshape(2, 1, 32), (2, ACC_ROWS, 32))

    s0 = _mm0(x, W0)
    h1 = _spmm(s0, gidx_f, didx_f, zinit, 128, jnp.float32, CHUNK_F32, NCH_F32)
    s1 = _mm1(h1, W1)
    h2 = _spmm(s1, gidx_f, didx_f, zinit, 128, jnp.float32, CHUNK_F32, NCH_F32)
    cs = _colsum_relu(h2)
    w2b = W2.reshape(2, 128, 2, 32).transpose(0, 2, 1, 3)
    s2 = _mm2(h2, cs, w2b)
    o = _spmm(s2, gidx_f, didx_f, binit, 32, jnp.float32, CHUNK_F32, NCH_F32)
    return jnp.concatenate([o[:N_NODES], o[N_NODES:]], axis=1)


# speedup vs baseline: 1.3074x; 1.0457x over previous
import functools

import jax
import jax.numpy as jnp
from jax import lax
from jax.experimental import pallas as pl
from jax.experimental.pallas import tpu as pltpu
from jax.experimental.pallas import tpu_sc as plsc

N_NODES = 10000
N_EDGES = 160000
NC, NS = 2, 16
NW = NC * NS
CHUNK_BF, NCH_BF = 128, 80
CHUNK_F32, NCH_F32 = 112, 90
ACC_ROWS = 10016
MBLK = 2000


def _spmm(table, gidx, didx, init, feat, dtype, chunk, nch):
    mesh = plsc.VectorSubcoreMesh(core_axis_name="c", subcore_axis_name="s")
    ri = ACC_ROWS // NS
    ro = N_NODES // NS
    NCH, CHUNK = nch, chunk

    @functools.partial(
        pl.kernel,
        out_type=jax.ShapeDtypeStruct((2 * N_NODES, feat), dtype),
        mesh=mesh,
        scratch_types=[
            pltpu.VMEM((NCH, CHUNK), jnp.int32),
            pltpu.VMEM((NCH, CHUNK), jnp.int32),
            pltpu.VMEM((CHUNK, feat), dtype),
            pltpu.VMEM((CHUNK, feat), dtype),
            pltpu.VMEM_SHARED((ACC_ROWS, feat), dtype),
            pltpu.SemaphoreType.DMA,
            pltpu.SemaphoreType.DMA,
            pltpu.SemaphoreType.DMA,
            pltpu.SemaphoreType.DMA,
        ],
        compiler_params=pltpu.CompilerParams(use_tc_tiling_on_sc=False),
    )
    def spmm(table_hbm, gidx_hbm, didx_hbm, init_hbm, out_hbm,
             gi_v, di_v, buf0, buf1, acc, semg0, semg1, sems0, sems1):
        c = lax.axis_index("c")
        s = lax.axis_index("s")
        pltpu.sync_copy(gidx_hbm.at[s], gi_v)
        pltpu.sync_copy(didx_hbm.at[s], di_v)
        pltpu.sync_copy(init_hbm.at[c, pl.ds(s * ri, ri)],
                        acc.at[pl.ds(s * ri, ri)])

        @pl.when(c == 1)
        def _():
            def bias_row(j, carry):
                for k in range(CHUNK // 16):
                    gi_v[j, pl.ds(k * 16, 16)] = (
                        gi_v[j, pl.ds(k * 16, 16)] + N_NODES)
                return carry
            lax.fori_loop(0, NCH, bias_row, 0)
        plsc.subcore_barrier()

        pltpu.async_copy(table_hbm.at[gi_v.at[0]], buf0, semg0)
        pltpu.async_copy(table_hbm.at[gi_v.at[1]], buf1, semg1)

        def body(i, carry):
            a = 2 * i
            pltpu.make_async_copy(table_hbm.at[gi_v.at[a]], buf0, semg0).wait()
            pltpu.sync_copy(buf0, acc.at[di_v.at[a]], add=True)

            @pl.when(i < NCH // 2 - 1)
            def _():
                pltpu.async_copy(table_hbm.at[gi_v.at[a + 2]], buf0, semg0)
            pltpu.make_async_copy(table_hbm.at[gi_v.at[a + 1]], buf1, semg1).wait()
            pltpu.sync_copy(buf1, acc.at[di_v.at[a + 1]], add=True)

            @pl.when(i < NCH // 2 - 1)
            def _():
                pltpu.async_copy(table_hbm.at[gi_v.at[a + 3]], buf1, semg1)
            return carry
        lax.fori_loop(0, NCH // 2, body, 0)
        plsc.subcore_barrier()

        pltpu.sync_copy(acc.at[pl.ds(s * ro, ro)],
                        out_hbm.at[pl.ds(c * N_NODES + s * ro, ro)])

    return spmm(table, gidx, didx, init)


def _mm0(x, w):
    def body(x_ref, w_ref, o_ref):
        o_ref[...] = jnp.dot(x_ref[...], w_ref[...],
                             preferred_element_type=jnp.float32)
    return pl.pallas_call(
        body,
        grid=(N_NODES // MBLK, 2),
        in_specs=[pl.BlockSpec((MBLK, 256), lambda i, j: (i, 0)),
                  pl.BlockSpec((256, 128), lambda i, j: (0, j))],
        out_specs=pl.BlockSpec((MBLK, 128), lambda i, j: (j * (N_NODES // MBLK) + i, 0)),
        out_shape=jax.ShapeDtypeStruct((2 * N_NODES, 128), jnp.float32),
    )(x, w)


def _mm1(hst, w):
    nb = N_NODES // MBLK

    def body(hlo_ref, hhi_ref, w_ref, o_ref):
        hlo = jnp.maximum(hlo_ref[...].astype(jnp.float32), 0.0)
        hhi = jnp.maximum(hhi_ref[...].astype(jnp.float32), 0.0)
        part = (jnp.dot(hlo, w_ref[:128], preferred_element_type=jnp.float32)
                + jnp.dot(hhi, w_ref[128:], preferred_element_type=jnp.float32))
        o_ref[...] = part

    return pl.pallas_call(
        body,
        grid=(nb, 2),
        in_specs=[pl.BlockSpec((MBLK, 128), lambda i, j: (i, 0)),
                  pl.BlockSpec((MBLK, 128), lambda i, j: (nb + i, 0)),
                  pl.BlockSpec((256, 128), lambda i, j: (0, j))],
        out_specs=pl.BlockSpec((MBLK, 128), lambda i, j: (j * nb + i, 0)),
        out_shape=jax.ShapeDtypeStruct((2 * N_NODES, 128), jnp.float32),
    )(hst, hst, w)


def _colsum_relu(hst):
    nb = N_NODES // MBLK

    def body(a_ref, b_ref, o_ref):
        i = pl.program_id(0)
        a = jnp.maximum(a_ref[...].astype(jnp.float32), 0.0)
        b = jnp.maximum(b_ref[...].astype(jnp.float32), 0.0)
        sa = jnp.sum(a, axis=0, keepdims=True)
        sb = jnp.sum(b, axis=0, keepdims=True)
        part = jnp.concatenate([sa, sb], axis=0)

        @pl.when(i == 0)
        def _():
            o_ref[...] = part

        @pl.when(i != 0)
        def _():
            o_ref[...] += part

    return pl.pallas_call(
        body,
        grid=(nb,),
        in_specs=[pl.BlockSpec((MBLK, 128), lambda i: (i, 0)),
                  pl.BlockSpec((MBLK, 128), lambda i: (nb + i, 0))],
        out_specs=pl.BlockSpec((2, 128), lambda i: (0, 0)),
        out_shape=jax.ShapeDtypeStruct((2, 128), jnp.float32),
    )(hst, hst)


def _mm2(hst, csum, w2b):
    nb = N_NODES // MBLK

    def body(hlo_ref, hhi_ref, c_ref, wlo_ref, whi_ref, o_ref):
        m = c_ref[...] * (1.0 / N_NODES)
        hlo = jnp.maximum(hlo_ref[...].astype(jnp.float32), 0.0) + m[0:1]
        hhi = jnp.maximum(hhi_ref[...].astype(jnp.float32), 0.0) + m[1:2]
        o_ref[...] = (
            jnp.dot(hlo, wlo_ref[0, 0], preferred_element_type=jnp.float32)
            + jnp.dot(hhi, whi_ref[0, 0], preferred_element_type=jnp.float32))

    return pl.pallas_call(
        body,
        grid=(nb, 2),
        in_specs=[pl.BlockSpec((MBLK, 128), lambda i, j: (i, 0)),
                  pl.BlockSpec((MBLK, 128), lambda i, j: (nb + i, 0)),
                  pl.BlockSpec((2, 128), lambda i, j: (0, 0)),
                  pl.BlockSpec((1, 1, 128, 32), lambda i, j: (0, j, 0, 0)),
                  pl.BlockSpec((1, 1, 128, 32), lambda i, j: (1, j, 0, 0))],
        out_specs=pl.BlockSpec((MBLK, 32), lambda i, j: (j * nb + i, 0)),
        out_shape=jax.ShapeDtypeStruct((2 * N_NODES, 32), jnp.float32),
    )(hst, hst, csum, w2b, w2b)


def _edge_layout(src, dst, chunk, nch):
    e_pad = NS * nch * chunk
    pad = e_pad - N_EDGES
    src_p = jnp.concatenate([src, jnp.zeros((pad,), jnp.int32)])
    dst_p = jnp.concatenate([dst, jnp.full((pad,), N_NODES, jnp.int32)])
    return src_p.reshape(NS, nch, chunk), dst_p.reshape(NS, nch, chunk)


def kernel(x, edge_index, W0, W1, W2, b2):
    src = edge_index[0].astype(jnp.int32)
    dst = edge_index[1].astype(jnp.int32)
    gidx_f, didx_f = _edge_layout(src, dst, CHUNK_F32, NCH_F32)
    zinit = jnp.zeros((2, ACC_ROWS, 128), jnp.float32)
    binit = jnp.broadcast_to(b2.reshape(2, 1, 32), (2, ACC_ROWS, 32))

    s0 = _mm0(x, W0)
    h1 = _spmm(s0, gidx_f, didx_f, zinit, 128, jnp.float32, CHUNK_F32, NCH_F32)
    s1 = _mm1(h1, W1)
    h2 = _spmm(s1, gidx_f, didx_f, zinit, 128, jnp.float32, CHUNK_F32, NCH_F32)
    cs = _colsum_relu(h2)
    w2b = W2.reshape(2, 128, 2, 32).transpose(0, 2, 1, 3)
    s2 = _mm2(h2, cs, w2b)
    o = _spmm(s2, gidx_f, didx_f, binit, 32, jnp.float32, CHUNK_F32, NCH_F32)
    return jnp.concatenate([o[:N_NODES], o[N_NODES:]], axis=1)


# speedup vs baseline: 1.3362x; 1.0220x over previous
import functools

import jax
import jax.numpy as jnp
from jax import lax
from jax.experimental import pallas as pl
from jax.experimental.pallas import tpu as pltpu
from jax.experimental.pallas import tpu_sc as plsc

N_NODES = 10000
N_EDGES = 160000
NC, NS = 2, 16
NW = NC * NS
CHUNK_BF, NCH_BF = 128, 80
CHUNK_F32, NCH_F32 = 112, 90
ACC_ROWS = 10016
MBLK = 5000


def _spmm(table, gidx, didx, init, feat, dtype, chunk, nch):
    mesh = plsc.VectorSubcoreMesh(core_axis_name="c", subcore_axis_name="s")
    ri = ACC_ROWS // NS
    ro = N_NODES // NS
    NCH, CHUNK = nch, chunk

    @functools.partial(
        pl.kernel,
        out_type=jax.ShapeDtypeStruct((2 * N_NODES, feat), dtype),
        mesh=mesh,
        scratch_types=[
            pltpu.VMEM((NCH, CHUNK), jnp.int32),
            pltpu.VMEM((NCH, CHUNK), jnp.int32),
            pltpu.VMEM((CHUNK, feat), dtype),
            pltpu.VMEM((CHUNK, feat), dtype),
            pltpu.VMEM_SHARED((ACC_ROWS, feat), dtype),
            pltpu.SemaphoreType.DMA,
            pltpu.SemaphoreType.DMA,
            pltpu.SemaphoreType.DMA,
            pltpu.SemaphoreType.DMA,
        ],
        compiler_params=pltpu.CompilerParams(use_tc_tiling_on_sc=False),
    )
    def spmm(table_hbm, gidx_hbm, didx_hbm, init_hbm, out_hbm,
             gi_v, di_v, buf0, buf1, acc, semg0, semg1, sems0, sems1):
        c = lax.axis_index("c")
        s = lax.axis_index("s")
        pltpu.sync_copy(gidx_hbm.at[s], gi_v)
        pltpu.sync_copy(didx_hbm.at[s], di_v)
        pltpu.sync_copy(init_hbm.at[c, pl.ds(s * ri, ri)],
                        acc.at[pl.ds(s * ri, ri)])

        @pl.when(c == 1)
        def _():
            def bias_row(j, carry):
                for k in range(CHUNK // 16):
                    gi_v[j, pl.ds(k * 16, 16)] = (
                        gi_v[j, pl.ds(k * 16, 16)] + N_NODES)
                return carry
            lax.fori_loop(0, NCH, bias_row, 0)
        plsc.subcore_barrier()

        pltpu.async_copy(table_hbm.at[gi_v.at[0]], buf0, semg0)
        pltpu.async_copy(table_hbm.at[gi_v.at[1]], buf1, semg1)

        def body(i, carry):
            a = 2 * i
            pltpu.make_async_copy(table_hbm.at[gi_v.at[a]], buf0, semg0).wait()
            pltpu.sync_copy(buf0, acc.at[di_v.at[a]], add=True)

            @pl.when(i < NCH // 2 - 1)
            def _():
                pltpu.async_copy(table_hbm.at[gi_v.at[a + 2]], buf0, semg0)
            pltpu.make_async_copy(table_hbm.at[gi_v.at[a + 1]], buf1, semg1).wait()
            pltpu.sync_copy(buf1, acc.at[di_v.at[a + 1]], add=True)

            @pl.when(i < NCH // 2 - 1)
            def _():
                pltpu.async_copy(table_hbm.at[gi_v.at[a + 3]], buf1, semg1)
            return carry
        lax.fori_loop(0, NCH // 2, body, 0)
        plsc.subcore_barrier()

        pltpu.sync_copy(acc.at[pl.ds(s * ro, ro)],
                        out_hbm.at[pl.ds(c * N_NODES + s * ro, ro)])

    return spmm(table, gidx, didx, init)


def _mm0(x, w):
    def body(x_ref, w_ref, o_ref):
        o_ref[...] = jnp.dot(x_ref[...], w_ref[...],
                             preferred_element_type=jnp.float32)
    return pl.pallas_call(
        body,
        grid=(N_NODES // MBLK, 2),
        in_specs=[pl.BlockSpec((MBLK, 256), lambda i, j: (i, 0)),
                  pl.BlockSpec((256, 128), lambda i, j: (0, j))],
        out_specs=pl.BlockSpec((MBLK, 128), lambda i, j: (j * (N_NODES // MBLK) + i, 0)),
        out_shape=jax.ShapeDtypeStruct((2 * N_NODES, 128), jnp.float32),
    )(x, w)


def _mm1(hst, w):
    nb = N_NODES // MBLK

    def body(hlo_ref, hhi_ref, w_ref, o_ref):
        hlo = jnp.maximum(hlo_ref[...].astype(jnp.float32), 0.0)
        hhi = jnp.maximum(hhi_ref[...].astype(jnp.float32), 0.0)
        part = (jnp.dot(hlo, w_ref[:128], preferred_element_type=jnp.float32)
                + jnp.dot(hhi, w_ref[128:], preferred_element_type=jnp.float32))
        o_ref[...] = part

    return pl.pallas_call(
        body,
        grid=(nb, 2),
        in_specs=[pl.BlockSpec((MBLK, 128), lambda i, j: (i, 0)),
                  pl.BlockSpec((MBLK, 128), lambda i, j: (nb + i, 0)),
                  pl.BlockSpec((256, 128), lambda i, j: (0, j))],
        out_specs=pl.BlockSpec((MBLK, 128), lambda i, j: (j * nb + i, 0)),
        out_shape=jax.ShapeDtypeStruct((2 * N_NODES, 128), jnp.float32),
    )(hst, hst, w)


def _colsum_relu(hst):
    nb = N_NODES // MBLK

    def body(a_ref, b_ref, o_ref):
        i = pl.program_id(0)
        a = jnp.maximum(a_ref[...].astype(jnp.float32), 0.0)
        b = jnp.maximum(b_ref[...].astype(jnp.float32), 0.0)
        sa = jnp.sum(a, axis=0, keepdims=True)
        sb = jnp.sum(b, axis=0, keepdims=True)
        part = jnp.concatenate([sa, sb], axis=0)

        @pl.when(i == 0)
        def _():
            o_ref[...] = part

        @pl.when(i != 0)
        def _():
            o_ref[...] += part

    return pl.pallas_call(
        body,
        grid=(nb,),
        in_specs=[pl.BlockSpec((MBLK, 128), lambda i: (i, 0)),
                  pl.BlockSpec((MBLK, 128), lambda i: (nb + i, 0))],
        out_specs=pl.BlockSpec((2, 128), lambda i: (0, 0)),
        out_shape=jax.ShapeDtypeStruct((2, 128), jnp.float32),
    )(hst, hst)


def _mm2(hst, csum, w2b):
    nb = N_NODES // MBLK

    def body(hlo_ref, hhi_ref, c_ref, wlo_ref, whi_ref, o_ref):
        m = c_ref[...] * (1.0 / N_NODES)
        hlo = jnp.maximum(hlo_ref[...].astype(jnp.float32), 0.0) + m[0:1]
        hhi = jnp.maximum(hhi_ref[...].astype(jnp.float32), 0.0) + m[1:2]
        o_ref[...] = (
            jnp.dot(hlo, wlo_ref[0, 0], preferred_element_type=jnp.float32)
            + jnp.dot(hhi, whi_ref[0, 0], preferred_element_type=jnp.float32))

    return pl.pallas_call(
        body,
        grid=(nb, 2),
        in_specs=[pl.BlockSpec((MBLK, 128), lambda i, j: (i, 0)),
                  pl.BlockSpec((MBLK, 128), lambda i, j: (nb + i, 0)),
                  pl.BlockSpec((2, 128), lambda i, j: (0, 0)),
                  pl.BlockSpec((1, 1, 128, 32), lambda i, j: (0, j, 0, 0)),
                  pl.BlockSpec((1, 1, 128, 32), lambda i, j: (1, j, 0, 0))],
        out_specs=pl.BlockSpec((MBLK, 32), lambda i, j: (j * nb + i, 0)),
        out_shape=jax.ShapeDtypeStruct((2 * N_NODES, 32), jnp.float32),
    )(hst, hst, csum, w2b, w2b)


def _edge_layout(src, dst, chunk, nch):
    e_pad = NS * nch * chunk
    pad = e_pad - N_EDGES
    src_p = jnp.concatenate([src, jnp.zeros((pad,), jnp.int32)])
    dst_p = jnp.concatenate([dst, jnp.full((pad,), N_NODES, jnp.int32)])
    return src_p.reshape(NS, nch, chunk), dst_p.reshape(NS, nch, chunk)


def kernel(x, edge_index, W0, W1, W2, b2):
    src = edge_index[0].astype(jnp.int32)
    dst = edge_index[1].astype(jnp.int32)
    gidx_f, didx_f = _edge_layout(src, dst, CHUNK_F32, NCH_F32)
    zinit = jnp.zeros((2, ACC_ROWS, 128), jnp.float32)
    binit = jnp.broadcast_to(b2.reshape(2, 1, 32), (2, ACC_ROWS, 32))

    s0 = _mm0(x, W0)
    h1 = _spmm(s0, gidx_f, didx_f, zinit, 128, jnp.float32, CHUNK_F32, NCH_F32)
    s1 = _mm1(h1, W1)
    h2 = _spmm(s1, gidx_f, didx_f, zinit, 128, jnp.float32, CHUNK_F32, NCH_F32)
    cs = _colsum_relu(h2)
    w2b = W2.reshape(2, 128, 2, 32).transpose(0, 2, 1, 3)
    s2 = _mm2(h2, cs, w2b)
    o = _spmm(s2, gidx_f, didx_f, binit, 32, jnp.float32, CHUNK_F32, NCH_F32)
    return jnp.concatenate([o[:N_NODES], o[N_NODES:]], axis=1)


# speedup vs baseline: 1.3580x; 1.0164x over previous
import functools

import jax
import jax.numpy as jnp
from jax import lax
from jax.experimental import pallas as pl
from jax.experimental.pallas import tpu as pltpu
from jax.experimental.pallas import tpu_sc as plsc

N_NODES = 10000
N_EDGES = 160000
NC, NS = 2, 16
NW = NC * NS
CHUNK_BF, NCH_BF = 128, 80
CHUNK_F32, NCH_F32 = 112, 90
ACC_ROWS = 10016
MBLK = 10000


def _spmm(table, gidx, didx, init, feat, dtype, chunk, nch):
    mesh = plsc.VectorSubcoreMesh(core_axis_name="c", subcore_axis_name="s")
    ri = ACC_ROWS // NS
    ro = N_NODES // NS
    NCH, CHUNK = nch, chunk

    @functools.partial(
        pl.kernel,
        out_type=jax.ShapeDtypeStruct((2 * N_NODES, feat), dtype),
        mesh=mesh,
        scratch_types=[
            pltpu.VMEM((NCH, CHUNK), jnp.int32),
            pltpu.VMEM((NCH, CHUNK), jnp.int32),
            pltpu.VMEM((CHUNK, feat), dtype),
            pltpu.VMEM((CHUNK, feat), dtype),
            pltpu.VMEM_SHARED((ACC_ROWS, feat), dtype),
            pltpu.SemaphoreType.DMA,
            pltpu.SemaphoreType.DMA,
            pltpu.SemaphoreType.DMA,
            pltpu.SemaphoreType.DMA,
        ],
        compiler_params=pltpu.CompilerParams(use_tc_tiling_on_sc=False),
    )
    def spmm(table_hbm, gidx_hbm, didx_hbm, init_hbm, out_hbm,
             gi_v, di_v, buf0, buf1, acc, semg0, semg1, sems0, sems1):
        c = lax.axis_index("c")
        s = lax.axis_index("s")
        pltpu.sync_copy(gidx_hbm.at[s], gi_v)
        pltpu.sync_copy(didx_hbm.at[s], di_v)
        pltpu.sync_copy(init_hbm.at[c, pl.ds(s * ri, ri)],
                        acc.at[pl.ds(s * ri, ri)])

        @pl.when(c == 1)
        def _():
            def bias_row(j, carry):
                for k in range(CHUNK // 16):
                    gi_v[j, pl.ds(k * 16, 16)] = (
                        gi_v[j, pl.ds(k * 16, 16)] + N_NODES)
                return carry
            lax.fori_loop(0, NCH, bias_row, 0)
        plsc.subcore_barrier()

        pltpu.async_copy(table_hbm.at[gi_v.at[0]], buf0, semg0)
        pltpu.async_copy(table_hbm.at[gi_v.at[1]], buf1, semg1)

        def body(i, carry):
            a = 2 * i
            pltpu.make_async_copy(table_hbm.at[gi_v.at[a]], buf0, semg0).wait()
            pltpu.sync_copy(buf0, acc.at[di_v.at[a]], add=True)

            @pl.when(i < NCH // 2 - 1)
            def _():
                pltpu.async_copy(table_hbm.at[gi_v.at[a + 2]], buf0, semg0)
            pltpu.make_async_copy(table_hbm.at[gi_v.at[a + 1]], buf1, semg1).wait()
            pltpu.sync_copy(buf1, acc.at[di_v.at[a + 1]], add=True)

            @pl.when(i < NCH // 2 - 1)
            def _():
                pltpu.async_copy(table_hbm.at[gi_v.at[a + 3]], buf1, semg1)
            return carry
        lax.fori_loop(0, NCH // 2, body, 0)
        plsc.subcore_barrier()

        pltpu.sync_copy(acc.at[pl.ds(s * ro, ro)],
                        out_hbm.at[pl.ds(c * N_NODES + s * ro, ro)])

    return spmm(table, gidx, didx, init)


def _mm0(x, w):
    def body(x_ref, w_ref, o_ref):
        o_ref[...] = jnp.dot(x_ref[...], w_ref[...],
                             preferred_element_type=jnp.float32)
    return pl.pallas_call(
        body,
        grid=(N_NODES // MBLK, 2),
        in_specs=[pl.BlockSpec((MBLK, 256), lambda i, j: (i, 0)),
                  pl.BlockSpec((256, 128), lambda i, j: (0, j))],
        out_specs=pl.BlockSpec((MBLK, 128), lambda i, j: (j * (N_NODES // MBLK) + i, 0)),
        out_shape=jax.ShapeDtypeStruct((2 * N_NODES, 128), jnp.float32),
    )(x, w)


def _mm1(hst, w):
    nb = N_NODES // MBLK

    def body(hlo_ref, hhi_ref, w_ref, o_ref):
        hlo = jnp.maximum(hlo_ref[...].astype(jnp.float32), 0.0)
        hhi = jnp.maximum(hhi_ref[...].astype(jnp.float32), 0.0)
        part = (jnp.dot(hlo, w_ref[:128], preferred_element_type=jnp.float32)
                + jnp.dot(hhi, w_ref[128:], preferred_element_type=jnp.float32))
        o_ref[...] = part

    return pl.pallas_call(
        body,
        grid=(nb, 2),
        in_specs=[pl.BlockSpec((MBLK, 128), lambda i, j: (i, 0)),
                  pl.BlockSpec((MBLK, 128), lambda i, j: (nb + i, 0)),
                  pl.BlockSpec((256, 128), lambda i, j: (0, j))],
        out_specs=pl.BlockSpec((MBLK, 128), lambda i, j: (j * nb + i, 0)),
        out_shape=jax.ShapeDtypeStruct((2 * N_NODES, 128), jnp.float32),
    )(hst, hst, w)


def _colsum_relu(hst):
    nb = N_NODES // MBLK

    def body(a_ref, b_ref, o_ref):
        i = pl.program_id(0)
        a = jnp.maximum(a_ref[...].astype(jnp.float32), 0.0)
        b = jnp.maximum(b_ref[...].astype(jnp.float32), 0.0)
        sa = jnp.sum(a, axis=0, keepdims=True)
        sb = jnp.sum(b, axis=0, keepdims=True)
        part = jnp.concatenate([sa, sb], axis=0)

        @pl.when(i == 0)
        def _():
            o_ref[...] = part

        @pl.when(i != 0)
        def _():
            o_ref[...] += part

    return pl.pallas_call(
        body,
        grid=(nb,),
        in_specs=[pl.BlockSpec((MBLK, 128), lambda i: (i, 0)),
                  pl.BlockSpec((MBLK, 128), lambda i: (nb + i, 0))],
        out_specs=pl.BlockSpec((2, 128), lambda i: (0, 0)),
        out_shape=jax.ShapeDtypeStruct((2, 128), jnp.float32),
    )(hst, hst)


def _mm2(hst, csum, w2b):
    nb = N_NODES // MBLK

    def body(hlo_ref, hhi_ref, c_ref, wlo_ref, whi_ref, o_ref):
        m = c_ref[...] * (1.0 / N_NODES)
        hlo = jnp.maximum(hlo_ref[...].astype(jnp.float32), 0.0) + m[0:1]
        hhi = jnp.maximum(hhi_ref[...].astype(jnp.float32), 0.0) + m[1:2]
        o_ref[...] = (
            jnp.dot(hlo, wlo_ref[0, 0], preferred_element_type=jnp.float32)
            + jnp.dot(hhi, whi_ref[0, 0], preferred_element_type=jnp.float32))

    return pl.pallas_call(
        body,
        grid=(nb, 2),
        in_specs=[pl.BlockSpec((MBLK, 128), lambda i, j: (i, 0)),
                  pl.BlockSpec((MBLK, 128), lambda i, j: (nb + i, 0)),
                  pl.BlockSpec((2, 128), lambda i, j: (0, 0)),
                  pl.BlockSpec((1, 1, 128, 32), lambda i, j: (0, j, 0, 0)),
                  pl.BlockSpec((1, 1, 128, 32), lambda i, j: (1, j, 0, 0))],
        out_specs=pl.BlockSpec((MBLK, 32), lambda i, j: (j * nb + i, 0)),
        out_shape=jax.ShapeDtypeStruct((2 * N_NODES, 32), jnp.float32),
    )(hst, hst, csum, w2b, w2b)


def _edge_layout(src, dst, chunk, nch):
    e_pad = NS * nch * chunk
    pad = e_pad - N_EDGES
    src_p = jnp.concatenate([src, jnp.zeros((pad,), jnp.int32)])
    dst_p = jnp.concatenate([dst, jnp.full((pad,), N_NODES, jnp.int32)])
    return src_p.reshape(NS, nch, chunk), dst_p.reshape(NS, nch, chunk)


def kernel(x, edge_index, W0, W1, W2, b2):
    src = edge_index[0].astype(jnp.int32)
    dst = edge_index[1].astype(jnp.int32)
    gidx_f, didx_f = _edge_layout(src, dst, CHUNK_F32, NCH_F32)
    zinit = jnp.zeros((2, ACC_ROWS, 128), jnp.float32)
    binit = jnp.broadcast_to(b2.reshape(2, 1, 32), (2, ACC_ROWS, 32))

    s0 = _mm0(x, W0)
    h1 = _spmm(s0, gidx_f, didx_f, zinit, 128, jnp.float32, CHUNK_F32, NCH_F32)
    s1 = _mm1(h1, W1)
    h2 = _spmm(s1, gidx_f, didx_f, zinit, 128, jnp.float32, CHUNK_F32, NCH_F32)
    cs = _colsum_relu(h2)
    w2b = W2.reshape(2, 128, 2, 32).transpose(0, 2, 1, 3)
    s2 = _mm2(h2, cs, w2b)
    o = _spmm(s2, gidx_f, didx_f, binit, 32, jnp.float32, CHUNK_F32, NCH_F32)
    return jnp.concatenate([o[:N_NODES], o[N_NODES:]], axis=1)
